# baseline (speedup 1.0000x reference)
import jax
import jax.numpy as jnp
from jax.experimental import pallas as pl

BATCH = 64
DIM_CODES = 128
DICT_SIZE = 8192
EMBED_DIM = 32
CT = 8


def _vq_kernel(mu_ref, dict_ref, z_ref, oh_ref):
    j = pl.program_id(1)
    mu = mu_ref[:, 0, j, :]
    d = dict_ref[0]
    cross = jax.lax.dot_general(
        mu, d, (((1,), (1,)), ((), ())),
        preferred_element_type=jnp.float32)
    mu_sq = jnp.sum(mu * mu, axis=1)
    d_sq = jnp.sum(d * d, axis=1)
    dist = mu_sq[:, None] + d_sq[None, :] - 2.0 * cross
    idx = jnp.argmin(dist, axis=1)
    iota = jax.lax.broadcasted_iota(jnp.int32, (BATCH, DICT_SIZE), 1)
    oh = (iota == idx[:, None]).astype(jnp.float32)
    oh_ref[:, 0, j, :] = oh
    z_ref[:, 0, j, :] = jax.lax.dot_general(
        oh, d, (((1,), (0,)), ((), ())),
        preferred_element_type=jnp.float32,
        precision=jax.lax.Precision.HIGHEST)


def kernel(mu, dictionary):
    B, C, K, E = BATCH, DIM_CODES, DICT_SIZE, EMBED_DIM
    G = C // CT
    mu4 = mu.reshape(B, G, CT, E)
    z4, oh4 = pl.pallas_call(
        _vq_kernel,
        grid=(G, CT),
        in_specs=[
            pl.BlockSpec((B, 1, CT, E), lambda c, j: (0, c, 0, 0)),
            pl.BlockSpec((1, K, E), lambda c, j: (c * CT + j, 0, 0)),
        ],
        out_specs=[
            pl.BlockSpec((B, 1, CT, E), lambda c, j: (0, c, 0, 0)),
            pl.BlockSpec((B, 1, CT, K), lambda c, j: (0, c, 0, 0)),
        ],
        out_shape=[
            jax.ShapeDtypeStruct((B, G, CT, E), jnp.float32),
            jax.ShapeDtypeStruct((B, G, CT, K), jnp.float32),
        ],
    )(mu4, dictionary)
    z = z4.reshape(B, C * E)
    return (z, z, oh4.reshape(B, C, K))


# speedup vs baseline: 1.4858x; 1.4858x over previous
import jax
import jax.numpy as jnp
from jax.experimental import pallas as pl
from jax.experimental.pallas import tpu as pltpu

BATCH = 64
DIM_CODES = 128
DICT_SIZE = 8192
EMBED_DIM = 32
CT = 8


def _vq_kernel(mu_ref, dict_hbm, z_ref, oh_ref, dbuf, sem):
    c0 = pl.program_id(0)
    row0 = c0 * CT

    @pl.when(c0 == 0)
    def _prologue():
        pltpu.make_async_copy(
            dict_hbm.at[0], dbuf.at[0], sem.at[0]).start()

    idx_rows = []
    for jr in range(CT):
        row = row0 + jr
        slot = jr % 2
        nxt_slot = (jr + 1) % 2
        @pl.when(row + 1 < DIM_CODES)
        def _prefetch():
            pltpu.make_async_copy(
                dict_hbm.at[row + 1], dbuf.at[nxt_slot],
                sem.at[nxt_slot]).start()
        pltpu.make_async_copy(
            dict_hbm.at[row], dbuf.at[slot], sem.at[slot]).wait()

        mu = mu_ref[jr]
        d = dbuf[slot]
        cross = jax.lax.dot_general(
            mu, d, (((1,), (1,)), ((), ())),
            preferred_element_type=jnp.float32)
        mu_sq = jnp.sum(mu * mu, axis=1)
        d_sq = jnp.sum(d * d, axis=1)
        dist = mu_sq[:, None] + d_sq[None, :] - 2.0 * cross
        idx = jnp.argmin(dist, axis=1)
        idx_rows.append(idx)
        oh = (jax.lax.broadcasted_iota(jnp.int32, (BATCH, DICT_SIZE), 1)
              == idx[:, None]).astype(jnp.float32)
        z_ref[jr] = jax.lax.dot_general(
            oh, d, (((1,), (0,)), ((), ())),
            preferred_element_type=jnp.float32)

    idx_all = jnp.stack(idx_rows, axis=1)
    iota3 = jax.lax.broadcasted_iota(jnp.int32, (BATCH, 1, CT, DICT_SIZE), 3)
    oh_ref[...] = (iota3 == idx_all[:, None, :, None]).astype(jnp.float32)


def kernel(mu, dictionary):
    B, C, K, E = BATCH, DIM_CODES, DICT_SIZE, EMBED_DIM
    G = C // CT
    mu_t = mu.reshape(B, C, E).transpose(1, 0, 2)
    z_t, oh4 = pl.pallas_call(
        _vq_kernel,
        grid=(G,),
        in_specs=[
            pl.BlockSpec((CT, B, E), lambda c: (c, 0, 0)),
            pl.BlockSpec(memory_space=pltpu.MemorySpace.HBM),
        ],
        out_specs=[
            pl.BlockSpec((CT, B, E), lambda c: (c, 0, 0)),
            pl.BlockSpec((B, 1, CT, K), lambda c: (0, c, 0, 0)),
        ],
        out_shape=[
            jax.ShapeDtypeStruct((C, B, E), jnp.float32),
            jax.ShapeDtypeStruct((B, G, CT, K), jnp.float32),
        ],
        scratch_shapes=[
            pltpu.VMEM((2, K, E), jnp.float32),
            pltpu.SemaphoreType.DMA((2,)),
        ],
    )(mu_t, dictionary)
    z = z_t.transpose(1, 0, 2).reshape(B, C * E)
    return (z, z, oh4.reshape(B, C, K))


# speedup vs baseline: 6.6120x; 4.4502x over previous
import jax
import jax.numpy as jnp
from jax.experimental import pallas as pl

BATCH = 64
DIM_CODES = 128
DICT_SIZE = 8192
EMBED_DIM = 32
CT = 8


def _vq_kernel(mu_ref, dict_ref, z_ref, oh_ref):
    iota = jax.lax.broadcasted_iota(jnp.int32, (BATCH, DICT_SIZE), 1)
    idx_rows = []
    for jr in range(CT):
        mu = mu_ref[jr]
        dT = dict_ref[jr]
        cross = jax.lax.dot_general(
            mu, dT, (((1,), (0,)), ((), ())),
            preferred_element_type=jnp.float32)
        mu_sq = jnp.sum(mu * mu, axis=1)
        d_sq = jnp.sum(dT * dT, axis=0)
        dist = mu_sq[:, None] + d_sq[None, :] - 2.0 * cross
        idx = jnp.argmin(dist, axis=1)
        idx_rows.append(idx)
        oh = (iota == idx[:, None]).astype(jnp.float32)
        z_ref[jr] = jax.lax.dot_general(
            oh, dT, (((1,), (1,)), ((), ())),
            preferred_element_type=jnp.float32)

    idx_all = jnp.stack(idx_rows, axis=1)
    iota4 = jax.lax.broadcasted_iota(jnp.int32, (BATCH, 1, CT, DICT_SIZE), 3)
    oh_ref[...] = (iota4 == idx_all[:, None, :, None]).astype(jnp.float32)


def kernel(mu, dictionary):
    B, C, K, E = BATCH, DIM_CODES, DICT_SIZE, EMBED_DIM
    G = C // CT
    dict_t = dictionary.transpose(0, 2, 1)
    mu_t = mu.reshape(B, C, E).transpose(1, 0, 2)
    z_t, oh4 = pl.pallas_call(
        _vq_kernel,
        grid=(G,),
        in_specs=[
            pl.BlockSpec((CT, B, E), lambda c: (c, 0, 0)),
            pl.BlockSpec((CT, E, K), lambda c: (c, 0, 0)),
        ],
        out_specs=[
            pl.BlockSpec((CT, B, E), lambda c: (c, 0, 0)),
            pl.BlockSpec((B, 1, CT, K), lambda c: (0, c, 0, 0)),
        ],
        out_shape=[
            jax.ShapeDtypeStruct((C, B, E), jnp.float32),
            jax.ShapeDtypeStruct((B, G, CT, K), jnp.float32),
        ],
    )(mu_t, dict_t)
    z = z_t.transpose(1, 0, 2).reshape(B, C * E)
    return (z, z, oh4.reshape(B, C, K))


# speedup vs baseline: 6.6513x; 1.0059x over previous
import jax
import jax.numpy as jnp
from jax.experimental import pallas as pl

BATCH = 64
DIM_CODES = 128
DICT_SIZE = 8192
EMBED_DIM = 32
CT = 8


def _vq_kernel(mu_ref, dict_ref, z_ref, oh_ref):
    iota = jax.lax.broadcasted_iota(jnp.int32, (BATCH, DICT_SIZE), 1)
    idx_rows = []
    z_rows = []
    for jr in range(CT):
        mu = mu_ref[:, jr * EMBED_DIM:(jr + 1) * EMBED_DIM]
        dT = dict_ref[jr]
        cross = jax.lax.dot_general(
            mu, dT, (((1,), (0,)), ((), ())),
            preferred_element_type=jnp.float32)
        mu_sq = jnp.sum(mu * mu, axis=1)
        d_sq = jnp.sum(dT * dT, axis=0)
        dist = mu_sq[:, None] + d_sq[None, :] - 2.0 * cross
        idx = jnp.argmin(dist, axis=1)
        idx_rows.append(idx)
        oh = (iota == idx[:, None]).astype(jnp.float32)
        z_rows.append(jax.lax.dot_general(
            oh, dT, (((1,), (1,)), ((), ())),
            preferred_element_type=jnp.float32))

    z_ref[...] = jnp.concatenate(z_rows, axis=1)
    idx_all = jnp.stack(idx_rows, axis=1)
    iota4 = jax.lax.broadcasted_iota(jnp.int32, (BATCH, 1, CT, DICT_SIZE), 3)
    oh_ref[...] = (iota4 == idx_all[:, None, :, None]).astype(jnp.float32)


def kernel(mu, dictionary):
    B, C, K, E = BATCH, DIM_CODES, DICT_SIZE, EMBED_DIM
    G = C // CT
    dict_t = dictionary.transpose(0, 2, 1)
    z, oh4 = pl.pallas_call(
        _vq_kernel,
        grid=(G,),
        in_specs=[
            pl.BlockSpec((B, CT * E), lambda c: (0, c)),
            pl.BlockSpec((CT, E, K), lambda c: (c, 0, 0)),
        ],
        out_specs=[
            pl.BlockSpec((B, CT * E), lambda c: (0, c)),
            pl.BlockSpec((B, 1, CT, K), lambda c: (0, c, 0, 0)),
        ],
        out_shape=[
            jax.ShapeDtypeStruct((B, C * E), jnp.float32),
            jax.ShapeDtypeStruct((B, G, CT, K), jnp.float32),
        ],
    )(mu, dict_t)
    return (z, z, oh4.reshape(B, C, K))
